# baseline (device time: 10982 ns/iter reference)
import jax
import jax.numpy as jnp
from jax import lax
from jax.experimental import pallas as pl
from jax.experimental.pallas import tpu as pltpu

N_DEV = 4
CHUNKS = 8


def kernel(x):
    m, n = x.shape
    chunk = m // CHUNKS
    x = pltpu.with_memory_space_constraint(x, pltpu.HBM)

    def body(x_hbm, out_ref, buf, comm_ref, copy_sems, send_sems, recv_sems):
        my_pos = lax.axis_index("i")

        def chunk_copy(g, slot):
            return pltpu.make_async_copy(
                x_hbm.at[pl.ds(g * chunk, chunk), :],
                buf.at[slot],
                copy_sems.at[slot],
            )

        chunk_copy(0, 0).start()
        barrier_sem = pltpu.get_barrier_semaphore()
        for k in range(1, N_DEV):
            peer = (my_pos + k) % N_DEV
            pl.semaphore_signal(
                barrier_sem, inc=1,
                device_id=(peer,), device_id_type=pl.DeviceIdType.MESH,
            )

        for g in range(CHUNKS):
            slot = g % 2
            if g + 1 < CHUNKS:
                chunk_copy(g + 1, (g + 1) % 2).start()
            chunk_copy(g, slot).wait()
            cm = jnp.max(buf[slot], axis=0, keepdims=True)
            if g == 0:
                comm_ref[0, :, :] = cm
            else:
                comm_ref[0, :, :] = jnp.maximum(comm_ref[0, :, :], cm)

        pl.semaphore_wait(barrier_sem, N_DEV - 1)

        rdmas = []
        for k in range(1, N_DEV):
            peer = (my_pos + k) % N_DEV
            rdma = pltpu.make_async_remote_copy(
                src_ref=comm_ref.at[0],
                dst_ref=comm_ref.at[k],
                send_sem=send_sems.at[k - 1],
                recv_sem=recv_sems.at[k - 1],
                device_id=(peer,),
                device_id_type=pl.DeviceIdType.MESH,
            )
            rdma.start()
            rdmas.append(rdma)
        for rdma in rdmas:
            rdma.wait()

        acc = comm_ref[0, :, :]
        for k in range(1, N_DEV):
            acc = jnp.maximum(acc, comm_ref[k, :, :])
        out_ref[:, :] = acc

    return pl.pallas_call(
        body,
        out_shape=jax.ShapeDtypeStruct((1, n), x.dtype),
        in_specs=[pl.BlockSpec(memory_space=pltpu.HBM)],
        out_specs=pl.BlockSpec(memory_space=pltpu.VMEM),
        scratch_shapes=[
            pltpu.VMEM((2, chunk, n), x.dtype),
            pltpu.VMEM((N_DEV, 1, n), x.dtype),
            pltpu.SemaphoreType.DMA((2,)),
            pltpu.SemaphoreType.DMA((N_DEV - 1,)),
            pltpu.SemaphoreType.DMA((N_DEV - 1,)),
        ],
        compiler_params=pltpu.CompilerParams(collective_id=0),
    )(x)


# device time: 8821 ns/iter; 1.2450x vs baseline; 1.2450x over previous
import jax
import jax.numpy as jnp
from jax import lax
from jax.experimental import pallas as pl
from jax.experimental.pallas import tpu as pltpu

N_DEV = 4
CHUNKS = 4


def kernel(x):
    m, n = x.shape
    chunk = m // CHUNKS
    x = pltpu.with_memory_space_constraint(x, pltpu.HBM)

    def body(x_hbm, out_ref, buf, comm_ref, copy_sems, send_sems, recv_sems):
        my_pos = lax.axis_index("i")

        def cp(g):
            return pltpu.make_async_copy(
                x_hbm.at[pl.ds(g * chunk, chunk), :],
                buf.at[g],
                copy_sems.at[g],
            )

        for g in range(CHUNKS):
            cp(g).start()
        barrier_sem = pltpu.get_barrier_semaphore()
        for k in range(1, N_DEV):
            peer = (my_pos + k) % N_DEV
            pl.semaphore_signal(
                barrier_sem, inc=1,
                device_id=(peer,), device_id_type=pl.DeviceIdType.MESH,
            )

        for g in range(CHUNKS):
            cp(g).wait()
            cm = jnp.max(buf[g], axis=0, keepdims=True)
            if g == 0:
                comm_ref[0, :, :] = cm
            else:
                comm_ref[0, :, :] = jnp.maximum(comm_ref[0, :, :], cm)

        pl.semaphore_wait(barrier_sem, N_DEV - 1)

        rdmas = []
        for k in range(1, N_DEV):
            peer = (my_pos + k) % N_DEV
            rdma = pltpu.make_async_remote_copy(
                src_ref=comm_ref.at[0],
                dst_ref=comm_ref.at[k],
                send_sem=send_sems.at[k - 1],
                recv_sem=recv_sems.at[k - 1],
                device_id=(peer,),
                device_id_type=pl.DeviceIdType.MESH,
            )
            rdma.start()
            rdmas.append(rdma)
        for rdma in rdmas:
            rdma.wait()

        acc = comm_ref[0, :, :]
        for k in range(1, N_DEV):
            acc = jnp.maximum(acc, comm_ref[k, :, :])
        out_ref[:, :] = acc

    return pl.pallas_call(
        body,
        out_shape=jax.ShapeDtypeStruct((1, n), x.dtype),
        in_specs=[pl.BlockSpec(memory_space=pltpu.HBM)],
        out_specs=pl.BlockSpec(memory_space=pltpu.VMEM),
        scratch_shapes=[
            pltpu.VMEM((CHUNKS, chunk, n), x.dtype),
            pltpu.VMEM((N_DEV, 1, n), x.dtype),
            pltpu.SemaphoreType.DMA((CHUNKS,)),
            pltpu.SemaphoreType.DMA((N_DEV - 1,)),
            pltpu.SemaphoreType.DMA((N_DEV - 1,)),
        ],
        compiler_params=pltpu.CompilerParams(collective_id=0),
    )(x)


# device time: 8772 ns/iter; 1.2519x vs baseline; 1.0056x over previous
import jax
import jax.numpy as jnp
from jax import lax
from jax.experimental import pallas as pl
from jax.experimental.pallas import tpu as pltpu

N_DEV = 4
CHUNKS = 8


def kernel(x):
    m, n = x.shape
    chunk = m // CHUNKS
    x = pltpu.with_memory_space_constraint(x, pltpu.HBM)

    def body(x_hbm, out_ref, buf, comm_ref, copy_sems, send_sems, recv_sems):
        my_pos = lax.axis_index("i")

        def cp(g):
            return pltpu.make_async_copy(
                x_hbm.at[pl.ds(g * chunk, chunk), :],
                buf.at[g],
                copy_sems.at[g],
            )

        for g in range(CHUNKS):
            cp(g).start()
        barrier_sem = pltpu.get_barrier_semaphore()
        for k in range(1, N_DEV):
            peer = (my_pos + k) % N_DEV
            pl.semaphore_signal(
                barrier_sem, inc=1,
                device_id=(peer,), device_id_type=pl.DeviceIdType.MESH,
            )

        for g in range(CHUNKS):
            cp(g).wait()
            cm = jnp.max(buf[g], axis=0, keepdims=True)
            if g == 0:
                comm_ref[0, :, :] = cm
            else:
                comm_ref[0, :, :] = jnp.maximum(comm_ref[0, :, :], cm)

        pl.semaphore_wait(barrier_sem, N_DEV - 1)

        rdmas = []
        for k in range(1, N_DEV):
            peer = (my_pos + k) % N_DEV
            rdma = pltpu.make_async_remote_copy(
                src_ref=comm_ref.at[0],
                dst_ref=comm_ref.at[k],
                send_sem=send_sems.at[k - 1],
                recv_sem=recv_sems.at[k - 1],
                device_id=(peer,),
                device_id_type=pl.DeviceIdType.MESH,
            )
            rdma.start()
            rdmas.append(rdma)
        for rdma in rdmas:
            rdma.wait()

        acc = comm_ref[0, :, :]
        for k in range(1, N_DEV):
            acc = jnp.maximum(acc, comm_ref[k, :, :])
        out_ref[:, :] = acc

    return pl.pallas_call(
        body,
        out_shape=jax.ShapeDtypeStruct((1, n), x.dtype),
        in_specs=[pl.BlockSpec(memory_space=pltpu.HBM)],
        out_specs=pl.BlockSpec(memory_space=pltpu.VMEM),
        scratch_shapes=[
            pltpu.VMEM((CHUNKS, chunk, n), x.dtype),
            pltpu.VMEM((N_DEV, 1, n), x.dtype),
            pltpu.SemaphoreType.DMA((CHUNKS,)),
            pltpu.SemaphoreType.DMA((N_DEV - 1,)),
            pltpu.SemaphoreType.DMA((N_DEV - 1,)),
        ],
        compiler_params=pltpu.CompilerParams(collective_id=0),
    )(x)
